# baseline (speedup 1.0000x reference)
import functools

import jax
import jax.numpy as jnp
from jax import lax
from jax.experimental import pallas as pl
from jax.experimental.pallas import tpu as pltpu
from jax.experimental.pallas import tpu_sc as plsc

B = 16384
D = 32
NC = 2
NS = 16
NW = NC * NS
BPW = B // NW
C = 128
NCH = BPW // C
L = 16

NJ_A, NJ_C, NJ_T = 20, 4, 8
R0_A, R0_C, R0_T = 0, NJ_A * NCH, (NJ_A + NJ_C) * NCH
NROWS = (NJ_A + NJ_C + NJ_T) * NCH

H1, H2 = 64, 32


def _sc_gather_body(u_tab, m_tab, a_tab, c_tab, t_tab,
                    u_idx, m_idx, a_idx, c_idx, t_idx, zrows,
                    uo, mo, ao, co, to,
                    uv, mv, av, cv, tv, idxf, ua, ma, aa, ca, ta,
                    semi, semz, semum, sema):
  wid = lax.axis_index("s") * NC + lax.axis_index("c")
  base = wid * BPW

  di = [pltpu.async_copy(u_idx.at[pl.ds(base, BPW)], uv, semi),
        pltpu.async_copy(m_idx.at[pl.ds(base, BPW)], mv, semi),
        pltpu.async_copy(a_idx.at[pl.ds(base, BPW)], av, semi),
        pltpu.async_copy(c_idx.at[pl.ds(base, BPW)], cv, semi),
        pltpu.async_copy(t_idx.at[pl.ds(base, BPW)], tv, semi)]
  dz = [pltpu.async_copy(zrows, acc, semz) for acc in (aa, ca, ta)]
  for d in di:
    d.wait()

  dum = []
  for idx_s, tab, acc in ((uv, u_tab, ua), (mv, m_tab, ma)):
    for c in range(NCH):
      dum.append(pltpu.async_copy(tab.at[idx_s.at[pl.ds(c * C, C)]],
                                  acc.at[pl.ds(c * C, C)], semum))
  for d in dz:
    d.wait()

  iota = lax.iota(jnp.int32, L)

  def pooled(src_v, nj, tab, acc, r0):
    def body(j, carry):
      col = jnp.broadcast_to(j, (L,))
      for c in range(NCH):
        row = (r0 + j * NCH + c) * C
        for k0 in range(C // L):
          rows = c * C + k0 * L + iota
          vals = plsc.load_gather(src_v, [rows, col])
          idxf[pl.ds(pl.multiple_of(row + k0 * L, L), L)] = vals
        start = pl.multiple_of(row, C)
        pltpu.async_copy(tab.at[idxf.at[pl.ds(start, C)]],
                         acc.at[pl.ds(c * C, C)], sema, add=True)
      return carry
    lax.fori_loop(0, nj, body, 0)

  pooled(av, NJ_A, a_tab, aa, R0_A)
  pooled(cv, NJ_C, c_tab, ca, R0_C)
  pooled(tv, NJ_T, t_tab, ta, R0_T)

  def drain_body(i, carry):
    pltpu.make_async_copy(a_tab.at[pl.ds(0, BPW)], aa, sema).wait()
    return carry
  lax.fori_loop(0, NROWS // NCH, drain_body, 0)
  for d in dum:
    d.wait()

  for acc, out in ((ua, uo), (ma, mo), (aa, ao), (ca, co), (ta, to)):
    pltpu.sync_copy(acc, out.at[pl.ds(base, BPW)])


@functools.cache
def _sc_gather():
  mesh = plsc.VectorSubcoreMesh(core_axis_name="c", subcore_axis_name="s",
                                num_cores=NC, num_subcores=NS)
  return pl.kernel(
      _sc_gather_body,
      out_type=[jax.ShapeDtypeStruct((B, D), jnp.float32) for _ in range(5)],
      mesh=mesh,
      compiler_params=pltpu.CompilerParams(use_tc_tiling_on_sc=False,
                                           needs_layout_passes=False),
      scratch_types=[
          pltpu.VMEM((BPW,), jnp.int32),
          pltpu.VMEM((BPW,), jnp.int32),
          pltpu.VMEM((BPW, NJ_A), jnp.int32),
          pltpu.VMEM((BPW, NJ_C), jnp.int32),
          pltpu.VMEM((BPW, NJ_T), jnp.int32),
          pltpu.VMEM((NROWS * C,), jnp.int32),
          pltpu.VMEM((BPW, D), jnp.float32),
          pltpu.VMEM((BPW, D), jnp.float32),
          pltpu.VMEM((BPW, D), jnp.float32),
          pltpu.VMEM((BPW, D), jnp.float32),
          pltpu.VMEM((BPW, D), jnp.float32),
          pltpu.SemaphoreType.DMA,
          pltpu.SemaphoreType.DMA,
          pltpu.SemaphoreType.DMA,
          pltpu.SemaphoreType.DMA,
      ],
  )


BT = 2048


def _mlp_body(u, m, a, c, t, w1, b1, w2, b2, w3t, b3, o):
  h = jnp.dot(u[...], w1[0:D, :], preferred_element_type=jnp.float32)
  h += jnp.dot(m[...], w1[D:2 * D, :], preferred_element_type=jnp.float32)
  h += jnp.dot(a[...], w1[2 * D:3 * D, :], preferred_element_type=jnp.float32)
  h += jnp.dot(c[...], w1[3 * D:4 * D, :], preferred_element_type=jnp.float32)
  h += jnp.dot(t[...], w1[4 * D:5 * D, :], preferred_element_type=jnp.float32)
  h = jnp.maximum(h + b1[...], 0.0)
  h = jnp.maximum(jnp.dot(h, w2[...], preferred_element_type=jnp.float32)
                  + b2[...], 0.0)
  o[...] = jnp.sum(h * w3t[...], axis=1) + b3[0, 0]


def _mlp(ue, me, ae, ce, te, w1s, b1, w2, b2, w3t, b3):
  emb_spec = pl.BlockSpec((BT, D), lambda i: (i, 0))
  full = lambda *s: pl.BlockSpec(s, lambda i: tuple(0 for _ in s))
  return pl.pallas_call(
      _mlp_body,
      grid=(B // BT,),
      in_specs=[emb_spec] * 5 + [full(5 * D, H1), full(1, H1), full(H1, H2),
                                 full(1, H2), full(1, H2), full(1, 1)],
      out_specs=pl.BlockSpec((BT,), lambda i: (i,)),
      out_shape=jax.ShapeDtypeStruct((B,), jnp.float32),
  )(ue, me, ae, ce, te, w1s, b1, w2, b2, w3t, b3)


def kernel(user, movie, actor, country, movie_type,
           user_table, movie_table, actor_table, country_table, type_table,
           W1, b1, W2, b2, W3, b3):
  zrows = jnp.zeros((BPW, D), jnp.float32)
  ue, me, ae, ce, te = _sc_gather()(
      user_table, movie_table, actor_table, country_table, type_table,
      user.astype(jnp.int32), movie, actor, country, movie_type, zrows)
  scale = jnp.concatenate([
      jnp.ones((2 * D,), jnp.float32),
      jnp.full((D,), 1.0 / NJ_A, jnp.float32),
      jnp.full((D,), 1.0 / NJ_C, jnp.float32),
      jnp.full((D,), 1.0 / NJ_T, jnp.float32),
  ])[:, None]
  w1s = W1 * scale
  return _mlp(ue, me, ae, ce, te, w1s, b1.reshape(1, H1), W2,
              b2.reshape(1, H2), W3.reshape(1, H2), b3.reshape(1, 1))


# speedup vs baseline: 1.0065x; 1.0065x over previous
import functools

import jax
import jax.numpy as jnp
from jax import lax
from jax.experimental import pallas as pl
from jax.experimental.pallas import tpu as pltpu
from jax.experimental.pallas import tpu_sc as plsc

B = 16384
D = 32
NC = 2
NS = 16
NW = NC * NS
BPW = B // NW
C = 128
NCH = BPW // C
L = 16

NJ_A, NJ_C, NJ_T = 20, 4, 8
R0_A, R0_C, R0_T = 0, NJ_A * NCH, (NJ_A + NJ_C) * NCH
NROWS = (NJ_A + NJ_C + NJ_T) * NCH

H1, H2 = 64, 32


def _user_movie_body(ru, rm, g_u, g_m, c_u, c_m, uo, mo,
                     guv, gmv, cuv, cmv, sbu, sbm, accu, accm, sem):
  wid = lax.axis_index("s") * NC + lax.axis_index("c")
  base = wid * BPW
  pltpu.sync_copy(g_u.at[pl.ds(base, BPW)], guv)
  pltpu.sync_copy(g_m.at[pl.ds(base, BPW)], gmv)
  pltpu.sync_copy(c_u.at[pl.ds(base, BPW)], cuv)
  pltpu.sync_copy(c_m.at[pl.ds(base, BPW)], cmv)

  iota = lax.iota(jnp.int32, L)

  for ch in range(NCH):
    k0 = ch * C
    du = pltpu.async_copy(ru.at[guv.at[pl.ds(k0, C)]], sbu, sem)
    dm = pltpu.async_copy(rm.at[gmv.at[pl.ds(k0, C)]], sbm, sem)
    du.wait()
    dm.wait()

    for k16 in range(C // L):
      kvec = iota + k16 * L
      for cv, sb, acc in ((cuv, sbu, accu), (cmv, sbm, accm)):
        cvec = cv[pl.ds(k0 + k16 * L, L)] * D

        def jbody(j, carry):
          v = plsc.load_gather(sb, [kvec, cvec + j])
          acc[pl.ds(pl.multiple_of(j * BPW + k0 + k16 * L, L), L)] = v
          return carry
        lax.fori_loop(0, D, jbody, 0)

  for j in range(D):
    pltpu.sync_copy(accu.at[pl.ds(j * BPW, BPW)], uo.at[j, pl.ds(base, BPW)])
    pltpu.sync_copy(accm.at[pl.ds(j * BPW, BPW)], mo.at[j, pl.ds(base, BPW)])


@functools.cache
def _user_movie():
  mesh = plsc.VectorSubcoreMesh(core_axis_name="c", subcore_axis_name="s",
                                num_cores=NC, num_subcores=NS)
  return pl.kernel(
      _user_movie_body,
      out_type=[jax.ShapeDtypeStruct((D, B), jnp.float32),
                jax.ShapeDtypeStruct((D, B), jnp.float32)],
      mesh=mesh,
      compiler_params=pltpu.CompilerParams(use_tc_tiling_on_sc=True,
                                           needs_layout_passes=False),
      scratch_types=[
          pltpu.VMEM((BPW,), jnp.int32),
          pltpu.VMEM((BPW,), jnp.int32),
          pltpu.VMEM((BPW,), jnp.int32),
          pltpu.VMEM((BPW,), jnp.int32),
          pltpu.VMEM((C, 4 * D), jnp.float32),
          pltpu.VMEM((C, 4 * D), jnp.float32),
          pltpu.VMEM((D * BPW,), jnp.float32),
          pltpu.VMEM((D * BPW,), jnp.float32),
          pltpu.SemaphoreType.DMA,
      ],
  )


def _sc_gather_body(a_tab, c_tab, t_tab,
                    a_idx, c_idx, t_idx, zrows,
                    ao, co, to,
                    av, cv, tv, idxf, aa, ca, ta,
                    semi, semz, sema):
  wid = lax.axis_index("s") * NC + lax.axis_index("c")
  base = wid * BPW

  di = [pltpu.async_copy(a_idx.at[pl.ds(base, BPW)], av, semi),
        pltpu.async_copy(c_idx.at[pl.ds(base, BPW)], cv, semi),
        pltpu.async_copy(t_idx.at[pl.ds(base, BPW)], tv, semi)]
  dz = [pltpu.async_copy(zrows, acc, semz) for acc in (aa, ca, ta)]
  for d in di:
    d.wait()
  for d in dz:
    d.wait()

  iota = lax.iota(jnp.int32, L)

  def pooled(src_v, nj, tab, acc, r0):
    def body(j, carry):
      col = jnp.broadcast_to(j, (L,))
      for c in range(NCH):
        row = (r0 + j * NCH + c) * C
        for k0 in range(C // L):
          rows = c * C + k0 * L + iota
          vals = plsc.load_gather(src_v, [rows, col])
          idxf[pl.ds(pl.multiple_of(row + k0 * L, L), L)] = vals
        start = pl.multiple_of(row, C)
        pltpu.async_copy(tab.at[idxf.at[pl.ds(start, C)]],
                         acc.at[pl.ds(c * C, C)], sema, add=True)
      return carry
    lax.fori_loop(0, nj, body, 0)

  pooled(av, NJ_A, a_tab, aa, R0_A)
  pooled(cv, NJ_C, c_tab, ca, R0_C)
  pooled(tv, NJ_T, t_tab, ta, R0_T)

  def drain_body(i, carry):
    pltpu.make_async_copy(a_tab.at[pl.ds(0, BPW)], aa, sema).wait()
    return carry
  lax.fori_loop(0, NROWS // NCH, drain_body, 0)

  for acc, out in ((aa, ao), (ca, co), (ta, to)):
    pltpu.sync_copy(acc, out.at[pl.ds(base, BPW)])


@functools.cache
def _sc_gather():
  mesh = plsc.VectorSubcoreMesh(core_axis_name="c", subcore_axis_name="s",
                                num_cores=NC, num_subcores=NS)
  return pl.kernel(
      _sc_gather_body,
      out_type=[jax.ShapeDtypeStruct((B, D), jnp.float32) for _ in range(3)],
      mesh=mesh,
      compiler_params=pltpu.CompilerParams(use_tc_tiling_on_sc=False,
                                           needs_layout_passes=False),
      scratch_types=[
          pltpu.VMEM((BPW, NJ_A), jnp.int32),
          pltpu.VMEM((BPW, NJ_C), jnp.int32),
          pltpu.VMEM((BPW, NJ_T), jnp.int32),
          pltpu.VMEM((NROWS * C,), jnp.int32),
          pltpu.VMEM((BPW, D), jnp.float32),
          pltpu.VMEM((BPW, D), jnp.float32),
          pltpu.VMEM((BPW, D), jnp.float32),
          pltpu.SemaphoreType.DMA,
          pltpu.SemaphoreType.DMA,
          pltpu.SemaphoreType.DMA,
      ],
  )


BT = 2048


def _mlp_body(u, m, a, c, t, w1, b1, w2, b2, w3t, b3, o):
  tdot = lambda x, w: lax.dot_general(
      x, w, (((0,), (0,)), ((), ())), preferred_element_type=jnp.float32)
  h = tdot(u[...], w1[0:D, :])
  h += tdot(m[...], w1[D:2 * D, :])
  h += jnp.dot(a[...], w1[2 * D:3 * D, :], preferred_element_type=jnp.float32)
  h += jnp.dot(c[...], w1[3 * D:4 * D, :], preferred_element_type=jnp.float32)
  h += jnp.dot(t[...], w1[4 * D:5 * D, :], preferred_element_type=jnp.float32)
  h = jnp.maximum(h + b1[...], 0.0)
  h = jnp.maximum(jnp.dot(h, w2[...], preferred_element_type=jnp.float32)
                  + b2[...], 0.0)
  o[...] = jnp.sum(h * w3t[...], axis=1) + b3[0, 0]


def _mlp(ut, mt, ae, ce, te, w1s, b1, w2, b2, w3t, b3):
  t_spec = pl.BlockSpec((D, BT), lambda i: (0, i))
  emb_spec = pl.BlockSpec((BT, D), lambda i: (i, 0))
  full = lambda *s: pl.BlockSpec(s, lambda i: tuple(0 for _ in s))
  return pl.pallas_call(
      _mlp_body,
      grid=(B // BT,),
      in_specs=[t_spec, t_spec] + [emb_spec] * 3 + [
          full(5 * D, H1), full(1, H1), full(H1, H2),
          full(1, H2), full(1, H2), full(1, 1)],
      out_specs=pl.BlockSpec((BT,), lambda i: (i,)),
      out_shape=jax.ShapeDtypeStruct((B,), jnp.float32),
  )(ut, mt, ae, ce, te, w1s, b1, w2, b2, w3t, b3)


def kernel(user, movie, actor, country, movie_type,
           user_table, movie_table, actor_table, country_table, type_table,
           W1, b1, W2, b2, W3, b3):
  zrows = jnp.zeros((BPW, D), jnp.float32)
  ru = user_table.reshape(-1, 4 * D)
  rm = movie_table.reshape(-1, 4 * D)
  ui = user.astype(jnp.int32)
  ut, mt = _user_movie()(ru, rm, ui >> 2, movie >> 2, ui & 3, movie & 3)
  ae, ce, te = _sc_gather()(
      actor_table, country_table, type_table,
      actor, country, movie_type, zrows)
  scale = jnp.concatenate([
      jnp.ones((2 * D,), jnp.float32),
      jnp.full((D,), 1.0 / NJ_A, jnp.float32),
      jnp.full((D,), 1.0 / NJ_C, jnp.float32),
      jnp.full((D,), 1.0 / NJ_T, jnp.float32),
  ])[:, None]
  w1s = W1 * scale
  return _mlp(ut, mt, ae, ce, te, w1s, b1.reshape(1, H1), W2,
              b2.reshape(1, H2), W3.reshape(1, H2), b3.reshape(1, 1))


# speedup vs baseline: 1.0318x; 1.0251x over previous
import functools

import jax
import jax.numpy as jnp
from jax import lax
from jax.experimental import pallas as pl
from jax.experimental.pallas import tpu as pltpu
from jax.experimental.pallas import tpu_sc as plsc

B = 16384
D = 32
NC = 2
NS = 16
NW = NC * NS
BPW = B // NW
C = 128
NCH = BPW // C
L = 16

NJ_A, NJ_C, NJ_T = 20, 4, 8
R0_A, R0_C, R0_T = 0, NJ_A * NCH, (NJ_A + NJ_C) * NCH
NROWS = (NJ_A + NJ_C + NJ_T) * NCH

H1, H2 = 64, 32


def _user_movie_body(pu, pm, u_idx, m_idx, uo, mo, guv, gmv, sbu, sbm, sem):
  wid = lax.axis_index("s") * NC + lax.axis_index("c")
  base = wid * BPW
  pltpu.sync_copy(u_idx.at[pl.ds(base, BPW)], guv)
  pltpu.sync_copy(m_idx.at[pl.ds(base, BPW)], gmv)

  for ch in range(NCH):
    k0 = ch * C
    du = pltpu.async_copy(pu.at[guv.at[pl.ds(k0, C)]], sbu, sem)
    dm = pltpu.async_copy(pm.at[gmv.at[pl.ds(k0, C)]], sbm, sem)
    du.wait()
    dm.wait()
    pltpu.sync_copy(sbu, uo.at[pl.ds(base + k0, C)])
    pltpu.sync_copy(sbm, mo.at[pl.ds(base + k0, C)])


@functools.cache
def _user_movie():
  mesh = plsc.VectorSubcoreMesh(core_axis_name="c", subcore_axis_name="s",
                                num_cores=NC, num_subcores=NS)
  return pl.kernel(
      _user_movie_body,
      out_type=[jax.ShapeDtypeStruct((B, 128), jnp.float32),
                jax.ShapeDtypeStruct((B, 128), jnp.float32)],
      mesh=mesh,
      compiler_params=pltpu.CompilerParams(use_tc_tiling_on_sc=True,
                                           needs_layout_passes=False),
      scratch_types=[
          pltpu.VMEM((BPW,), jnp.int32),
          pltpu.VMEM((BPW,), jnp.int32),
          pltpu.VMEM((C, 128), jnp.float32),
          pltpu.VMEM((C, 128), jnp.float32),
          pltpu.SemaphoreType.DMA,
      ],
  )


def _sc_gather_body(a_tab, c_tab, t_tab,
                    a_idx, c_idx, t_idx, zrows,
                    ao, co, to,
                    av, cv, tv, idxf, aa, ca, ta,
                    semi, semz, sema):
  wid = lax.axis_index("s") * NC + lax.axis_index("c")
  base = wid * BPW

  di = [pltpu.async_copy(a_idx.at[pl.ds(base, BPW)], av, semi),
        pltpu.async_copy(c_idx.at[pl.ds(base, BPW)], cv, semi),
        pltpu.async_copy(t_idx.at[pl.ds(base, BPW)], tv, semi)]
  dz = [pltpu.async_copy(zrows, acc, semz) for acc in (aa, ca, ta)]
  for d in di:
    d.wait()
  for d in dz:
    d.wait()

  iota = lax.iota(jnp.int32, L)

  def pooled(src_v, nj, tab, acc, r0):
    def body(j, carry):
      col = jnp.broadcast_to(j, (L,))
      for c in range(NCH):
        row = (r0 + j * NCH + c) * C
        for k0 in range(C // L):
          rows = c * C + k0 * L + iota
          vals = plsc.load_gather(src_v, [rows, col])
          idxf[pl.ds(pl.multiple_of(row + k0 * L, L), L)] = vals
        start = pl.multiple_of(row, C)
        pltpu.async_copy(tab.at[idxf.at[pl.ds(start, C)]],
                         acc.at[pl.ds(c * C, C)], sema, add=True)
      return carry
    lax.fori_loop(0, nj, body, 0)

  pooled(av, NJ_A, a_tab, aa, R0_A)
  pooled(cv, NJ_C, c_tab, ca, R0_C)
  pooled(tv, NJ_T, t_tab, ta, R0_T)

  def drain_body(i, carry):
    pltpu.make_async_copy(a_tab.at[pl.ds(0, BPW)], aa, sema).wait()
    return carry
  lax.fori_loop(0, NROWS // NCH, drain_body, 0)

  for acc, out in ((aa, ao), (ca, co), (ta, to)):
    pltpu.sync_copy(acc, out.at[pl.ds(base, BPW)])


@functools.cache
def _sc_gather():
  mesh = plsc.VectorSubcoreMesh(core_axis_name="c", subcore_axis_name="s",
                                num_cores=NC, num_subcores=NS)
  return pl.kernel(
      _sc_gather_body,
      out_type=[jax.ShapeDtypeStruct((B, D), jnp.float32) for _ in range(3)],
      mesh=mesh,
      compiler_params=pltpu.CompilerParams(use_tc_tiling_on_sc=False,
                                           needs_layout_passes=False),
      scratch_types=[
          pltpu.VMEM((BPW, NJ_A), jnp.int32),
          pltpu.VMEM((BPW, NJ_C), jnp.int32),
          pltpu.VMEM((BPW, NJ_T), jnp.int32),
          pltpu.VMEM((NROWS * C,), jnp.int32),
          pltpu.VMEM((BPW, D), jnp.float32),
          pltpu.VMEM((BPW, D), jnp.float32),
          pltpu.VMEM((BPW, D), jnp.float32),
          pltpu.SemaphoreType.DMA,
          pltpu.SemaphoreType.DMA,
          pltpu.SemaphoreType.DMA,
      ],
  )


BT = 2048


def _mlp_body(u, m, a, c, t, w1u, w1m, w1r, b1, w2, b2, w3t, b3, o):
  h = jnp.dot(u[...], w1u[...], preferred_element_type=jnp.float32)
  h += jnp.dot(m[...], w1m[...], preferred_element_type=jnp.float32)
  h += jnp.dot(a[...], w1r[0:D, :], preferred_element_type=jnp.float32)
  h += jnp.dot(c[...], w1r[D:2 * D, :], preferred_element_type=jnp.float32)
  h += jnp.dot(t[...], w1r[2 * D:3 * D, :], preferred_element_type=jnp.float32)
  h = jnp.maximum(h + b1[...], 0.0)
  h = jnp.maximum(jnp.dot(h, w2[...], preferred_element_type=jnp.float32)
                  + b2[...], 0.0)
  o[...] = jnp.sum(h * w3t[...], axis=1) + b3[0, 0]


def _mlp(ue, me, ae, ce, te, w1u, w1m, w1r, b1, w2, b2, w3t, b3):
  pad_spec = pl.BlockSpec((BT, 128), lambda i: (i, 0))
  emb_spec = pl.BlockSpec((BT, D), lambda i: (i, 0))
  full = lambda *s: pl.BlockSpec(s, lambda i: tuple(0 for _ in s))
  return pl.pallas_call(
      _mlp_body,
      grid=(B // BT,),
      in_specs=[pad_spec, pad_spec] + [emb_spec] * 3 + [
          full(128, H1), full(128, H1), full(3 * D, H1), full(1, H1),
          full(H1, H2), full(1, H2), full(1, H2), full(1, 1)],
      out_specs=pl.BlockSpec((BT,), lambda i: (i,)),
      out_shape=jax.ShapeDtypeStruct((B,), jnp.float32),
  )(ue, me, ae, ce, te, w1u, w1m, w1r, b1, w2, b2, w3t, b3)


def kernel(user, movie, actor, country, movie_type,
           user_table, movie_table, actor_table, country_table, type_table,
           W1, b1, W2, b2, W3, b3):
  zrows = jnp.zeros((BPW, D), jnp.float32)
  pu = jnp.pad(user_table, ((0, 0), (0, 128 - D)))
  pm = jnp.pad(movie_table, ((0, 0), (0, 128 - D)))
  ue, me = _user_movie()(pu, pm, user.astype(jnp.int32), movie)
  ae, ce, te = _sc_gather()(
      actor_table, country_table, type_table,
      actor, country, movie_type, zrows)
  scale = jnp.concatenate([
      jnp.full((D,), 1.0 / NJ_A, jnp.float32),
      jnp.full((D,), 1.0 / NJ_C, jnp.float32),
      jnp.full((D,), 1.0 / NJ_T, jnp.float32),
  ])[:, None]
  w1r = W1[2 * D:] * scale
  w1u = jnp.pad(W1[0:D], ((0, 128 - D), (0, 0)))
  w1m = jnp.pad(W1[D:2 * D], ((0, 128 - D), (0, 0)))
  return _mlp(ue, me, ae, ce, te, w1u, w1m, w1r, b1.reshape(1, H1), W2,
              b2.reshape(1, H2), W3.reshape(1, H2), b3.reshape(1, 1))
